# Initial kernel scaffold; baseline (speedup 1.0000x reference)
import jax
import jax.numpy as jnp
from jax.experimental import pallas as pl


def kernel(input_ids, positions, spec_hidden, emb_table, enorm_w, hnorm_w, eh_proj_w, ln1_w, wq, wk, wv, wo, ln2_w, router_w, w_gate, w_up, w_down, final_ln_w):
    raise NotImplementedError("write your pallas kernel here")


# speedup vs baseline: 1.4075x; 1.4075x over previous
import functools

import jax
import jax.numpy as jnp
from jax import lax
from jax.experimental import pallas as pl
from jax.experimental.pallas import tpu as pltpu
from jax.experimental.pallas import tpu_sc as plsc

T = 2048
D = 1024
H = 16
DH = 64
E = 8
F = 256
EPS = 1e-6
HD = H * DH

BT = 256
BQ = 512


def _rms(x, w):
    var = jnp.mean(x * x, axis=-1, keepdims=True)
    return x * lax.rsqrt(var + EPS) * w


def _embed_gather(table, idx):
    info = plsc.get_sparse_core_info()
    nw = info.num_cores * info.num_subcores
    b_per_w = T // nw
    mesh = plsc.VectorSubcoreMesh(core_axis_name="c", subcore_axis_name="s")

    @functools.partial(
        pl.kernel,
        mesh=mesh,
        out_type=jax.ShapeDtypeStruct((T, D), jnp.float32),
        scratch_types=[
            pltpu.VMEM((b_per_w,), jnp.int32),
            pltpu.VMEM((b_per_w, D), jnp.float32),
            pltpu.SemaphoreType.DMA,
        ],
    )
    def gather_k(table_hbm, idx_hbm, out_hbm, idx_v, rows_v, sem):
        wid = lax.axis_index("s") * info.num_cores + lax.axis_index("c")
        base = wid * b_per_w
        pltpu.sync_copy(idx_hbm.at[pl.ds(base, b_per_w)], idx_v)
        pltpu.async_copy(table_hbm.at[idx_v], rows_v, sem).wait()
        pltpu.sync_copy(rows_v, out_hbm.at[pl.ds(base, b_per_w)])

    return gather_k(table, idx)


def _prelude_body(emb_ref, spec_ref, ehp_ref, wq_ref, wk_ref, wv_ref,
                  enw_ref, hnw_ref, ln1_ref,
                  res_ref, q_ref, k_ref, v_ref):
    en = _rms(emb_ref[...], enw_ref[...])
    hn = _rms(spec_ref[...], hnw_ref[...])
    x = (jnp.dot(en, ehp_ref[:D, :], preferred_element_type=jnp.float32)
         + jnp.dot(hn, ehp_ref[D:, :], preferred_element_type=jnp.float32))
    res_ref[...] = x
    hs = _rms(x, ln1_ref[...])
    q_ref[...] = jnp.dot(hs, wq_ref[...], preferred_element_type=jnp.float32)
    k_ref[...] = jnp.dot(hs, wk_ref[...], preferred_element_type=jnp.float32)
    v_ref[...] = jnp.dot(hs, wv_ref[...], preferred_element_type=jnp.float32)


def _prelude(emb, spec, ehp, wq, wk, wv, enw, hnw, ln1):
    grid = (T // BT,)
    tok = pl.BlockSpec((BT, D), lambda i: (i, 0))
    full = lambda shape: pl.BlockSpec(shape, lambda i: (0,) * len(shape))
    return pl.pallas_call(
        _prelude_body,
        grid=grid,
        in_specs=[tok, tok, full((2 * D, D)), full((D, HD)), full((D, HD)),
                  full((D, HD)), full((1, D)), full((1, D)), full((1, D))],
        out_specs=[tok, pl.BlockSpec((BT, HD), lambda i: (i, 0)),
                   pl.BlockSpec((BT, HD), lambda i: (i, 0)),
                   pl.BlockSpec((BT, HD), lambda i: (i, 0))],
        out_shape=[jax.ShapeDtypeStruct((T, D), jnp.float32),
                   jax.ShapeDtypeStruct((T, HD), jnp.float32),
                   jax.ShapeDtypeStruct((T, HD), jnp.float32),
                   jax.ShapeDtypeStruct((T, HD), jnp.float32)],
    )(emb, spec, ehp, wq, wk, wv, enw, hnw, ln1)


def _rope_apply(x, cos, sin):
    x1 = x[:, : DH // 2]
    x2 = x[:, DH // 2:]
    return jnp.concatenate([x1 * cos - x2 * sin, x2 * cos + x1 * sin], axis=1)


def _attn_body(q_ref, k_ref, v_ref, cq_ref, sq_ref, ck_ref, sk_ref, o_ref):
    iq = pl.program_id(1)
    cq, sq = cq_ref[...], sq_ref[...]
    ck, sk = ck_ref[...], sk_ref[...]
    rows = iq * BQ + lax.broadcasted_iota(jnp.int32, (BQ, T), 0)
    cols = lax.broadcasted_iota(jnp.int32, (BQ, T), 1)
    mask = cols <= rows
    outs = []
    for hh in range(2):
        q = _rope_apply(q_ref[:, hh * DH:(hh + 1) * DH], cq, sq)
        k = _rope_apply(k_ref[:, hh * DH:(hh + 1) * DH], ck, sk)
        s = lax.dot_general(q, k, (((1,), (1,)), ((), ())),
                            preferred_element_type=jnp.float32) * 0.125
        s = jnp.where(mask, s, -1e30)
        m = jnp.max(s, axis=-1, keepdims=True)
        p = jnp.exp(s - m)
        p = p / jnp.sum(p, axis=-1, keepdims=True)
        outs.append(jnp.dot(p, v_ref[:, hh * DH:(hh + 1) * DH],
                            preferred_element_type=jnp.float32))
    o_ref[...] = jnp.concatenate(outs, axis=1)


def _attention(q, k, v, cos, sin):
    grid = (H // 2, T // BQ)
    head2 = lambda arr_rows: None
    return pl.pallas_call(
        _attn_body,
        grid=grid,
        in_specs=[
            pl.BlockSpec((BQ, 2 * DH), lambda h, i: (i, h)),
            pl.BlockSpec((T, 2 * DH), lambda h, i: (0, h)),
            pl.BlockSpec((T, 2 * DH), lambda h, i: (0, h)),
            pl.BlockSpec((BQ, DH // 2), lambda h, i: (i, 0)),
            pl.BlockSpec((BQ, DH // 2), lambda h, i: (i, 0)),
            pl.BlockSpec((T, DH // 2), lambda h, i: (0, 0)),
            pl.BlockSpec((T, DH // 2), lambda h, i: (0, 0)),
        ],
        out_specs=pl.BlockSpec((BQ, 2 * DH), lambda h, i: (i, h)),
        out_shape=jax.ShapeDtypeStruct((T, HD), jnp.float32),
    )(q, k, v, cos, sin, cos, sin)


def _post_body(o_ref, x_ref, wo_ref, ln2_ref, rw_ref, wg_ref, wu_ref, wd_ref,
               fln_ref, out_ref):
    attn = jnp.dot(o_ref[...], wo_ref[...], preferred_element_type=jnp.float32)
    resid = x_ref[...] + attn
    hs = _rms(resid, ln2_ref[...])
    logits = jnp.dot(hs, rw_ref[...], preferred_element_type=jnp.float32)
    m = jnp.max(logits, axis=-1, keepdims=True)
    ex = jnp.exp(logits - m)
    probs = ex / jnp.sum(ex, axis=-1, keepdims=True)
    ii = lax.broadcasted_iota(jnp.int32, (BT, E), 1)
    t1 = jnp.max(probs, axis=-1, keepdims=True)
    a1 = jnp.min(jnp.where(probs == t1, ii, E), axis=-1, keepdims=True)
    sel1 = ii == a1
    p2 = jnp.where(sel1, -1.0, probs)
    t2 = jnp.max(p2, axis=-1, keepdims=True)
    a2 = jnp.min(jnp.where(p2 == t2, ii, E), axis=-1, keepdims=True)
    sel2 = ii == a2
    we = (jnp.where(sel1, t1, 0.0) + jnp.where(sel2, t2, 0.0)) / (t1 + t2)
    acc = jnp.zeros((BT, D), jnp.float32)
    for e in range(E):
        g = jnp.dot(hs, wg_ref[e], preferred_element_type=jnp.float32)
        u = jnp.dot(hs, wu_ref[e], preferred_element_type=jnp.float32)
        act = g / (1.0 + jnp.exp(-g)) * u
        acc = acc + we[:, e:e + 1] * jnp.dot(
            act, wd_ref[e], preferred_element_type=jnp.float32)
    out_ref[...] = _rms(resid + acc, fln_ref[...])


def _post(o, x, wo, ln2, rw, wg, wu, wd, fln):
    grid = (T // BT,)
    tokd = pl.BlockSpec((BT, D), lambda i: (i, 0))
    full = lambda shape: pl.BlockSpec(shape, lambda i: (0,) * len(shape))
    return pl.pallas_call(
        _post_body,
        grid=grid,
        in_specs=[pl.BlockSpec((BT, HD), lambda i: (i, 0)), tokd,
                  full((HD, D)), full((1, D)), full((D, E)),
                  full((E, D, F)), full((E, D, F)), full((E, F, D)),
                  full((1, D))],
        out_specs=tokd,
        out_shape=jax.ShapeDtypeStruct((T, D), jnp.float32),
    )(o, x, wo, ln2, rw, wg, wu, wd, fln)


def kernel(input_ids, positions, spec_hidden, emb_table, enorm_w, hnorm_w,
           eh_proj_w, ln1_w, wq, wk, wv, wo, ln2_w, router_w, w_gate, w_up,
           w_down, final_ln_w):
    ids = input_ids.astype(jnp.int32)
    emb = _embed_gather(emb_table, ids)

    inv = 1.0 / (10000.0 ** (jnp.arange(0, DH, 2, dtype=jnp.float32) / DH))
    ang = positions.astype(jnp.float32)[:, None] * inv[None, :]
    cos = jnp.cos(ang)
    sin = jnp.sin(ang)

    res, q, k, v = _prelude(emb, spec_hidden, eh_proj_w, wq, wk, wv,
                            enorm_w.reshape(1, D), hnorm_w.reshape(1, D),
                            ln1_w.reshape(1, D))
    o = _attention(q, k, v, cos, sin)
    return _post(o, res, wo, ln2_w.reshape(1, D), router_w, w_gate, w_up,
                 w_down, final_ln_w.reshape(1, D))


# speedup vs baseline: 1.7337x; 1.2318x over previous
import functools

import jax
import jax.numpy as jnp
from jax import lax
from jax.experimental import pallas as pl
from jax.experimental.pallas import tpu as pltpu
from jax.experimental.pallas import tpu_sc as plsc

T = 2048
D = 1024
H = 16
DH = 64
E = 8
F = 256
EPS = 1e-6
HD = H * DH

BT = 256
BQ = 512


def _rms(x, w):
    var = jnp.mean(x * x, axis=-1, keepdims=True)
    return x * lax.rsqrt(var + EPS) * w


def _embed_gather(table, idx):
    info = plsc.get_sparse_core_info()
    nw = info.num_cores * info.num_subcores
    b_per_w = T // nw
    mesh = plsc.VectorSubcoreMesh(core_axis_name="c", subcore_axis_name="s")

    @functools.partial(
        pl.kernel,
        mesh=mesh,
        out_type=jax.ShapeDtypeStruct((T, D), jnp.float32),
        scratch_types=[
            pltpu.VMEM((b_per_w,), jnp.int32),
            pltpu.VMEM((b_per_w, D), jnp.float32),
            pltpu.SemaphoreType.DMA,
        ],
    )
    def gather_k(table_hbm, idx_hbm, out_hbm, idx_v, rows_v, sem):
        wid = lax.axis_index("s") * info.num_cores + lax.axis_index("c")
        base = wid * b_per_w
        pltpu.sync_copy(idx_hbm.at[pl.ds(base, b_per_w)], idx_v)
        pltpu.async_copy(table_hbm.at[idx_v], rows_v, sem).wait()
        pltpu.sync_copy(rows_v, out_hbm.at[pl.ds(base, b_per_w)])

    return gather_k(table, idx)


def _prelude_body(emb_ref, spec_ref, ehp_ref, wq_ref, wqp_ref, wk_ref,
                  wkp_ref, wv_ref, cos_ref, sin_ref,
                  enw_ref, hnw_ref, ln1_ref,
                  res_ref, q_ref, k_ref, v_ref):
    en = _rms(emb_ref[...], enw_ref[...])
    hn = _rms(spec_ref[...], hnw_ref[...])
    x = (jnp.dot(en, ehp_ref[:D, :], preferred_element_type=jnp.float32)
         + jnp.dot(hn, ehp_ref[D:, :], preferred_element_type=jnp.float32))
    res_ref[...] = x
    hs = _rms(x, ln1_ref[...])
    cos, sin = cos_ref[...], sin_ref[...]
    q_ref[...] = (jnp.dot(hs, wq_ref[...], preferred_element_type=jnp.float32) * cos
                  + jnp.dot(hs, wqp_ref[...], preferred_element_type=jnp.float32) * sin)
    k_ref[...] = (jnp.dot(hs, wk_ref[...], preferred_element_type=jnp.float32) * cos
                  + jnp.dot(hs, wkp_ref[...], preferred_element_type=jnp.float32) * sin)
    v_ref[...] = jnp.dot(hs, wv_ref[...], preferred_element_type=jnp.float32)


def _prelude(emb, spec, ehp, wq, wqp, wk, wkp, wv, cos_t, sin_t, enw, hnw, ln1):
    grid = (T // BT,)
    tok = pl.BlockSpec((BT, D), lambda i: (i, 0))
    full = lambda shape: pl.BlockSpec(shape, lambda i: (0,) * len(shape))
    return pl.pallas_call(
        _prelude_body,
        grid=grid,
        in_specs=[tok, tok, full((2 * D, D)), full((D, HD)), full((D, HD)),
                  full((D, HD)), full((D, HD)), full((D, HD)),
                  tok, tok, full((1, D)), full((1, D)), full((1, D))],
        out_specs=[tok, pl.BlockSpec((BT, HD), lambda i: (i, 0)),
                   pl.BlockSpec((BT, HD), lambda i: (i, 0)),
                   pl.BlockSpec((BT, HD), lambda i: (i, 0))],
        out_shape=[jax.ShapeDtypeStruct((T, D), jnp.float32),
                   jax.ShapeDtypeStruct((T, HD), jnp.float32),
                   jax.ShapeDtypeStruct((T, HD), jnp.float32),
                   jax.ShapeDtypeStruct((T, HD), jnp.float32)],
    )(emb, spec, ehp, wq, wqp, wk, wkp, wv, cos_t, sin_t, enw, hnw, ln1)


BK = 512


def _attn_body(q_ref, k_ref, v_ref, o_ref):
    iq = pl.program_id(1)
    rows = iq * BQ + lax.broadcasted_iota(jnp.int32, (BQ, BK), 0)
    cols0 = lax.broadcasted_iota(jnp.int32, (BQ, BK), 1)
    outs = []
    for hh in range(2):
        q = q_ref[:, hh * DH:(hh + 1) * DH]

        def body(j, carry):
            m, l, acc = carry
            kc = k_ref[pl.ds(j * BK, BK), hh * DH:(hh + 1) * DH]
            vc = v_ref[pl.ds(j * BK, BK), hh * DH:(hh + 1) * DH]
            s = lax.dot_general(q, kc, (((1,), (1,)), ((), ())),
                                preferred_element_type=jnp.float32) * 0.125
            s = jnp.where(j * BK + cols0 <= rows, s, -1e30)
            m_new = jnp.maximum(m, jnp.max(s, axis=-1, keepdims=True))
            alpha = jnp.exp(m - m_new)
            p = jnp.exp(s - m_new)
            l_new = l * alpha + jnp.sum(p, axis=-1, keepdims=True)
            acc_new = acc * alpha + jnp.dot(p, vc,
                                            preferred_element_type=jnp.float32)
            return m_new, l_new, acc_new

        init = (jnp.full((BQ, 1), -1e30, jnp.float32),
                jnp.zeros((BQ, 1), jnp.float32),
                jnp.zeros((BQ, DH), jnp.float32))
        m, l, acc = lax.fori_loop(0, iq + 1, body, init)
        outs.append(acc / l)
    o_ref[...] = jnp.concatenate(outs, axis=1)


def _attention(q, k, v):
    grid = (H // 2, T // BQ)
    return pl.pallas_call(
        _attn_body,
        grid=grid,
        in_specs=[
            pl.BlockSpec((BQ, 2 * DH), lambda h, i: (i, h)),
            pl.BlockSpec((T, 2 * DH), lambda h, i: (0, h)),
            pl.BlockSpec((T, 2 * DH), lambda h, i: (0, h)),
        ],
        out_specs=pl.BlockSpec((BQ, 2 * DH), lambda h, i: (i, h)),
        out_shape=jax.ShapeDtypeStruct((T, HD), jnp.float32),
    )(q, k, v)


def _post_body(o_ref, x_ref, wo_ref, ln2_ref, rw_ref, wg_ref, wu_ref, wd_ref,
               fln_ref, out_ref):
    attn = jnp.dot(o_ref[...], wo_ref[...], preferred_element_type=jnp.float32)
    resid = x_ref[...] + attn
    hs = _rms(resid, ln2_ref[...])
    logits = jnp.dot(hs, rw_ref[...], preferred_element_type=jnp.float32)
    m = jnp.max(logits, axis=-1, keepdims=True)
    ex = jnp.exp(logits - m)
    probs = ex / jnp.sum(ex, axis=-1, keepdims=True)
    ii = lax.broadcasted_iota(jnp.int32, (BT, E), 1)
    t1 = jnp.max(probs, axis=-1, keepdims=True)
    a1 = jnp.min(jnp.where(probs == t1, ii, E), axis=-1, keepdims=True)
    sel1 = ii == a1
    p2 = jnp.where(sel1, -1.0, probs)
    t2 = jnp.max(p2, axis=-1, keepdims=True)
    a2 = jnp.min(jnp.where(p2 == t2, ii, E), axis=-1, keepdims=True)
    sel2 = ii == a2
    we = (jnp.where(sel1, t1, 0.0) + jnp.where(sel2, t2, 0.0)) / (t1 + t2)
    acc = jnp.zeros((BT, D), jnp.float32)
    for e in range(E):
        g = jnp.dot(hs, wg_ref[e], preferred_element_type=jnp.float32)
        u = jnp.dot(hs, wu_ref[e], preferred_element_type=jnp.float32)
        act = g / (1.0 + jnp.exp(-g)) * u
        acc = acc + we[:, e:e + 1] * jnp.dot(
            act, wd_ref[e], preferred_element_type=jnp.float32)
    out_ref[...] = _rms(resid + acc, fln_ref[...])


def _post(o, x, wo, ln2, rw, wg, wu, wd, fln):
    grid = (T // BT,)
    tokd = pl.BlockSpec((BT, D), lambda i: (i, 0))
    full = lambda shape: pl.BlockSpec(shape, lambda i: (0,) * len(shape))
    return pl.pallas_call(
        _post_body,
        grid=grid,
        in_specs=[pl.BlockSpec((BT, HD), lambda i: (i, 0)), tokd,
                  full((HD, D)), full((1, D)), full((D, E)),
                  full((E, D, F)), full((E, D, F)), full((E, F, D)),
                  full((1, D))],
        out_specs=tokd,
        out_shape=jax.ShapeDtypeStruct((T, D), jnp.float32),
    )(o, x, wo, ln2, rw, wg, wu, wd, fln)


def kernel(input_ids, positions, spec_hidden, emb_table, enorm_w, hnorm_w,
           eh_proj_w, ln1_w, wq, wk, wv, wo, ln2_w, router_w, w_gate, w_up,
           w_down, final_ln_w):
    ids = input_ids.astype(jnp.int32)
    emb = _embed_gather(emb_table, ids)

    inv = 1.0 / (10000.0 ** (jnp.arange(0, DH, 2, dtype=jnp.float32) / DH))
    ang = positions.astype(jnp.float32)[:, None] * inv[None, :]
    cos_t = jnp.tile(jnp.cos(ang), (1, 2 * H))
    sin_t = jnp.tile(jnp.sin(ang), (1, 2 * H))

    def perm(w):
        w4 = w.reshape(D, H, 2, DH // 2)
        return jnp.concatenate([-w4[:, :, 1], w4[:, :, 0]], axis=2).reshape(D, HD)

    res, q, k, v = _prelude(emb, spec_hidden, eh_proj_w, wq, perm(wq),
                            wk, perm(wk), wv, cos_t, sin_t,
                            enorm_w.reshape(1, D), hnorm_w.reshape(1, D),
                            ln1_w.reshape(1, D))
    o = _attention(q, k, v)
    return _post(o, res, wo, ln2_w.reshape(1, D), router_w, w_gate, w_up,
                 w_down, final_ln_w.reshape(1, D))
